# Initial kernel scaffold; baseline (speedup 1.0000x reference)
import jax
import jax.numpy as jnp
from jax.experimental import pallas as pl


def kernel(x, ragged_dense, row_lengths):
    raise NotImplementedError("write your pallas kernel here")


# speedup vs baseline: 5.7474x; 5.7474x over previous
import jax
import jax.numpy as jnp
from jax.experimental import pallas as pl
from jax.experimental.pallas import tpu as pltpu

_B, _C = 16, 4096


def _frames_tc_kernel(lens_ref, x_ref, g_ref, out_ref):
    iota = jax.lax.broadcasted_iota(jnp.int32, (1, _C), 1)

    def body(i, carry):
        s = lens_ref[i]
        shift = (_C - s) % _C
        rx = pltpu.roll(x_ref[pl.ds(i, 1), :], shift, 1)
        rg = pltpu.roll(g_ref[pl.ds(i, 1), :], shift, 1)
        out_ref[pl.ds(i, 1), :] = jnp.where(iota < _C - s, rx, rg)
        return carry

    jax.lax.fori_loop(0, _B, body, 0, unroll=True)


def kernel(x, ragged_dense, row_lengths):
    lens = row_lengths.astype(jnp.int32)
    out = pl.pallas_call(
        _frames_tc_kernel,
        out_shape=jax.ShapeDtypeStruct((_B, _C), jnp.float32),
        in_specs=[
            pl.BlockSpec(memory_space=pltpu.SMEM),
            pl.BlockSpec(memory_space=pltpu.VMEM),
            pl.BlockSpec(memory_space=pltpu.VMEM),
        ],
        out_specs=pl.BlockSpec(memory_space=pltpu.VMEM),
    )(lens, x, ragged_dense)
    return out, lens[:, None]
